# baseline (speedup 1.0000x reference)
import functools

import jax
import jax.numpy as jnp
from jax import lax
from jax.experimental import pallas as pl
from jax.experimental.pallas import tpu as pltpu
from jax.experimental.pallas import tpu_sc as plsc

N = 10000
NP = 10240
H = 8
C = 16
HC = 128
G = 64
E = 320000
NC = 2
NS = 16
NWRK = NC * NS
BE = 128
DUMMY = 10100
ROWS_PER_TILE = NP // NS
MAXTBL = NP * H

BEL = 512
EPW1 = 10240
EP1 = EPW1 * NWRK
EPW23 = 10752
EP23 = EPW23 * NWRK

_HIGHEST = lax.Precision.HIGHEST


def _dotH(a, b, dims):
    return lax.dot_general(a, b, (dims, ((), ())),
                           precision=_HIGHEST,
                           preferred_element_type=jnp.float32)


def _prep_common(xw, sel):
    return _dotH(xw, sel, ((1,), (0,)))


def _sel128(att):
    a = att.reshape(HC)
    k = jnp.arange(HC)
    oh = (k[:, None] // C == jnp.arange(HC)[None, :]).astype(jnp.float32)
    return oh * a[:, None]


def _prep_l1(xpad, W, s_sel, d_sel):
    BM = 1280
    def body(x_ref, w_ref, ss_ref, ds_ref, xw_ref, s_ref, d_ref):
        xw = _dotH(x_ref[...], w_ref[...], ((1,), (0,)))
        xw_ref[...] = xw
        s_ref[...] = _prep_common(xw, ss_ref[...])
        d_ref[...] = _prep_common(xw, ds_ref[...])
    return pl.pallas_call(
        body,
        grid=(NP // BM,),
        in_specs=[pl.BlockSpec((BM, HC), lambda i: (i, 0)),
                  pl.BlockSpec((HC, HC), lambda i: (0, 0)),
                  pl.BlockSpec((HC, HC), lambda i: (0, 0)),
                  pl.BlockSpec((HC, HC), lambda i: (0, 0))],
        out_specs=[pl.BlockSpec((BM, HC), lambda i: (i, 0)),
                   pl.BlockSpec((BM, HC), lambda i: (i, 0)),
                   pl.BlockSpec((BM, HC), lambda i: (i, 0))],
        out_shape=[jax.ShapeDtypeStruct((NP, HC), jnp.float32),
                   jax.ShapeDtypeStruct((NP, HC), jnp.float32),
                   jax.ShapeDtypeStruct((NP, HC), jnp.float32)],
    )(xpad, W, s_sel, d_sel)


def _epilogue_h(m_ref, d_ref, b_ref, i, BM):
    msum = m_ref[0] + m_ref[1]
    dfull = d_ref[0] + d_ref[1]
    hval = jax.nn.relu(msum / (dfull + 1e-16) + b_ref[...])
    row = jax.lax.broadcasted_iota(jnp.int32, (BM, HC), 0) + i * BM
    return jnp.where(row < N, hval, 0.0)


def _prep_l23(msgp, denp, bias2d, W, s_sel, d_sel):
    BM = 1280
    def body(m_ref, d_ref, b_ref, w_ref, ss_ref, ds_ref,
             xw_ref, s_ref, d16_ref):
        i = pl.program_id(0)
        hval = _epilogue_h(m_ref, d_ref, b_ref, i, BM)
        xw = _dotH(hval, w_ref[...], ((1,), (0,)))
        xw_ref[...] = xw
        s_ref[...] = _prep_common(xw, ss_ref[...])
        d16_ref[...] = _prep_common(xw, ds_ref[...])
    return pl.pallas_call(
        body,
        grid=(NP // BM,),
        in_specs=[pl.BlockSpec((2, BM, HC), lambda i: (0, i, 0)),
                  pl.BlockSpec((2, BM, HC), lambda i: (0, i, 0)),
                  pl.BlockSpec((1, HC), lambda i: (0, 0)),
                  pl.BlockSpec((HC, HC), lambda i: (0, 0)),
                  pl.BlockSpec((HC, HC), lambda i: (0, 0)),
                  pl.BlockSpec((HC, HC), lambda i: (0, 0))],
        out_specs=[pl.BlockSpec((BM, HC), lambda i: (i, 0)),
                   pl.BlockSpec((BM, HC), lambda i: (i, 0)),
                   pl.BlockSpec((BM, HC), lambda i: (i, 0))],
        out_shape=[jax.ShapeDtypeStruct((NP, HC), jnp.float32),
                   jax.ShapeDtypeStruct((NP, HC), jnp.float32),
                   jax.ShapeDtypeStruct((NP, HC), jnp.float32)],
    )(msgp, denp, bias2d, W, s_sel, d_sel)


def _max_reduce(tables3d):
    def body(t_ref, o_ref):
        m = jnp.max(t_ref[...], axis=0)
        o_ref[...] = jnp.where(m < -1e29, 0.0, m)
    BR = 8
    return pl.pallas_call(
        body,
        grid=(80 // BR,),
        in_specs=[pl.BlockSpec((NWRK, BR, 1024), lambda i: (0, i, 0))],
        out_specs=pl.BlockSpec((BR, 1024), lambda i: (i, 0)),
        out_shape=jax.ShapeDtypeStruct((80, 1024), jnp.float32),
    )(tables3d)


def _exp_tc(args16):
    EPL = args16.shape[0]
    BM = 4096
    k = jnp.arange(HC)

    def body(a_ref, e_ref, o_ref):
        ex = jnp.exp(a_ref[...])
        o_ref[...] = _dotH(ex, e_ref[...], ((1,), (0,)))

    expand = (jnp.arange(16)[:, None] == (k[None, :] // C)).astype(
        jnp.float32)
    return pl.pallas_call(
        body,
        grid=(EPL // BM,),
        in_specs=[pl.BlockSpec((BM, 16), lambda i: (i, 0)),
                  pl.BlockSpec((16, HC), lambda i: (0, 0))],
        out_specs=pl.BlockSpec((BM, HC), lambda i: (i, 0)),
        out_shape=jax.ShapeDtypeStruct((EPL, HC), jnp.float32),
    )(args16, expand)


def _mult_tc(a, b):
    EPL = a.shape[0]
    BM = 4096
    def body(a_ref, b_ref, o_ref):
        o_ref[...] = a_ref[...] * b_ref[...]
    return pl.pallas_call(
        body,
        grid=(EPL // BM,),
        in_specs=[pl.BlockSpec((BM, HC), lambda i: (i, 0)),
                  pl.BlockSpec((BM, HC), lambda i: (i, 0))],
        out_specs=pl.BlockSpec((BM, HC), lambda i: (i, 0)),
        out_shape=jax.ShapeDtypeStruct((EPL, HC), jnp.float32),
    )(a, b)


def _final(msgp, denp, bias2d, batch2d, fw1, fb1r, fw2):
    BM = 1280
    def body(m_ref, d_ref, b_ref, bt_ref, f1_ref, f1b_ref, f2_ref,
             gs_ref, o_ref):
        i = pl.program_id(0)
        hval = _epilogue_h(m_ref, d_ref, b_ref, i, BM)
        onehot = (bt_ref[...] == jnp.arange(G)[None, :]).astype(jnp.float32)
        part = _dotH(onehot, hval, ((0,), (0,)))
        cnt = _dotH(onehot, jnp.ones((BM, HC), jnp.float32), ((0,), (0,)))

        @pl.when(i == 0)
        def _():
            gs_ref[...] = jnp.zeros((2, G, HC), jnp.float32)

        gs_ref[0, :, :] = gs_ref[0, :, :] + part
        gs_ref[1, :, :] = gs_ref[1, :, :] + cnt

        @pl.when(i == NP // BM - 1)
        def _():
            ge = gs_ref[0, :, :] / jnp.maximum(gs_ref[1, :, :], 1.0)
            hid = jax.nn.relu(_dotH(ge, f1_ref[...], ((1,), (1,)))
                              + f1b_ref[...])
            out = _dotH(hid, f2_ref[...], ((1,), (1,)))
            o_ref[...] = jnp.broadcast_to(out, (G, HC))
    return pl.pallas_call(
        body,
        grid=(NP // BM,),
        in_specs=[pl.BlockSpec((2, BM, HC), lambda i: (0, i, 0)),
                  pl.BlockSpec((2, BM, HC), lambda i: (0, i, 0)),
                  pl.BlockSpec((1, HC), lambda i: (0, 0)),
                  pl.BlockSpec((BM, 1), lambda i: (i, 0)),
                  pl.BlockSpec((G, HC), lambda i: (0, 0)),
                  pl.BlockSpec((1, G), lambda i: (0, 0)),
                  pl.BlockSpec((1, G), lambda i: (0, 0))],
        out_specs=[pl.BlockSpec((2, G, HC), lambda i: (0, 0, 0)),
                   pl.BlockSpec((G, HC), lambda i: (0, 0))],
        out_shape=[jax.ShapeDtypeStruct((2, G, HC), jnp.float32),
                   jax.ShapeDtypeStruct((G, HC), jnp.float32)],
    )(msgp, denp, bias2d, batch2d, fw1, fb1r, fw2)


_MESH = plsc.VectorSubcoreMesh(core_axis_name="c", subcore_axis_name="s")


def _wid():
    return lax.axis_index("s") * NC + lax.axis_index("c")


def _zero_zbuf(z_ref):
    zv = jnp.zeros((16,), jnp.float32)
    def zrow(r, _):
        for j in range(8):
            z_ref[r, pl.ds(16 * j, 16)] = zv
        return 0
    lax.fori_loop(0, 128, zrow, 0)


def _sc_pass_a(epw, ep):
    @functools.partial(
        pl.kernel, mesh=_MESH,
        compiler_params=pltpu.CompilerParams(needs_layout_passes=False),
        out_type=[jax.ShapeDtypeStruct((ep * 16,), jnp.float32),
                  jax.ShapeDtypeStruct((NWRK, MAXTBL), jnp.float32)],
        scratch_types=[
            pltpu.VMEM((BE,), jnp.int32),
            pltpu.VMEM((BE + 16,), jnp.int32),
            pltpu.VMEM((BE + 16,), jnp.float32),
            pltpu.VMEM((BE, HC), jnp.float32),
            pltpu.VMEM((BE, HC), jnp.float32),
            pltpu.VMEM((BE * 16,), jnp.float32),
            pltpu.VMEM((16,), jnp.float32),
            pltpu.VMEM((MAXTBL + 16,), jnp.float32),
            pltpu.SemaphoreType.DMA,
            pltpu.SemaphoreType.DMA,
        ],
    )
    def k(srcx, dstx, eax, asrc_t, adst_t, wcv, alpha_out, tbl_out,
          srcv, dstv, eav, asg, adg, alb, wcvv, mtbl, sem1, sem2):
        wid = _wid()
        pltpu.sync_copy(wcv, wcvv)
        wv = wcvv[...]
        cv = jnp.where(lax.iota(jnp.int32, 16) < 8, 0.0, -1e30)

        neg = jnp.full((16,), -1e30, jnp.float32)
        def init(r, _):
            mtbl[pl.ds(r * 16, 16)] = neg
            return 0
        lax.fori_loop(0, (MAXTBL + 16) // 16, init, 0)

        lane = lax.iota(jnp.int32, 16)
        spill = MAXTBL + lane

        def blk(g, _):
            base = wid * epw + g * BE
            pltpu.sync_copy(srcx.at[pl.ds(base, BE)], srcv)
            pltpu.sync_copy(dstx.at[pl.ds(base, BE)],
                            dstv.at[pl.ds(0, BE)])
            pltpu.sync_copy(eax.at[pl.ds(base, BE)], eav.at[pl.ds(0, BE)])
            cp1 = pltpu.async_copy(asrc_t.at[srcv], asg, sem1)
            cp2 = pltpu.async_copy(adst_t.at[dstv.at[pl.ds(0, BE)]], adg,
                                   sem2)
            cp1.wait()
            cp2.wait()

            def edge(b, _):
                ea = eav[pl.ds(b, 16)][0]
                a = ((asg[b, pl.ds(0, 16)] + adg[b, pl.ds(0, 16)])
                     + (ea * wv + cv))
                a = jnp.maximum(a, 0.0) + 0.2 * jnp.minimum(a, 0.0)
                alb[pl.ds(b * 16, 16)] = a
                d0 = dstv[pl.ds(b, 16)][0]
                idx = jnp.where(lane < 8, d0 * H + lane, spill)
                cur = plsc.load_gather(mtbl, [idx])
                plsc.store_scatter(mtbl, [idx], jnp.maximum(cur, a))
                return 0
            lax.fori_loop(0, BE, edge, 0)
            pltpu.sync_copy(alb, alpha_out.at[pl.ds(base * 16, BE * 16)])
            return 0
        lax.fori_loop(0, epw // BE, blk, 0)
        pltpu.sync_copy(mtbl.at[pl.ds(0, MAXTBL)], tbl_out.at[wid])
    return k


def _sc_pass_b(epw, ep):
    @functools.partial(
        pl.kernel, mesh=_MESH,
        compiler_params=pltpu.CompilerParams(needs_layout_passes=False),
        out_type=jax.ShapeDtypeStruct((ep * 16,), jnp.float32),
        scratch_types=[
            pltpu.VMEM((BE + 16,), jnp.int32),
            pltpu.VMEM((BE * 16,), jnp.float32),
            pltpu.VMEM((MAXTBL,), jnp.float32),
        ],
    )
    def k(dstx, alpha_in, amax_f, args_out, dstv, alb, amv):
        wid = _wid()
        pltpu.sync_copy(amax_f, amv)
        lane7 = jnp.bitwise_and(lax.iota(jnp.int32, 16), 7)

        def blk(g, _):
            base = wid * epw + g * BE
            pltpu.sync_copy(dstx.at[pl.ds(base, BE)],
                            dstv.at[pl.ds(0, BE)])
            pltpu.sync_copy(alpha_in.at[pl.ds(base * 16, BE * 16)], alb)

            def edge(b, _):
                d0 = dstv[pl.ds(b, 16)][0]
                idx = d0 * H + lane7
                am = plsc.load_gather(amv, [idx])
                alb[pl.ds(b * 16, 16)] = alb[pl.ds(b * 16, 16)] - am
                return 0
            lax.fori_loop(0, BE, edge, 0)
            pltpu.sync_copy(alb, args_out.at[pl.ds(base * 16, BE * 16)])
            return 0
        lax.fori_loop(0, epw // BE, blk, 0)
    return k


def _sc_gather(epw, ep):
    @functools.partial(
        pl.kernel, mesh=_MESH,
        out_type=jax.ShapeDtypeStruct((ep, HC), jnp.float32),
        scratch_types=[
            pltpu.VMEM((BEL,), jnp.int32),
            pltpu.VMEM((BEL, HC), jnp.float32),
            pltpu.SemaphoreType.DMA,
        ],
    )
    def k(srcx, xw_t, out, srcv, xwg, sem):
        wid = _wid()

        def blk(g, _):
            base = wid * epw + g * BEL
            pltpu.sync_copy(srcx.at[pl.ds(base, BEL)], srcv)
            pltpu.async_copy(xw_t.at[srcv], xwg, sem).wait()
            pltpu.sync_copy(xwg, out.at[pl.ds(base, BEL)])
            return 0
        lax.fori_loop(0, epw // BEL, blk, 0)
    return k


def _sc_scatter_add(epw, ep):
    @functools.partial(
        pl.kernel, mesh=_MESH,
        out_type=jax.ShapeDtypeStruct((NC, NP, HC), jnp.float32),
        scratch_types=[
            pltpu.VMEM((BE,), jnp.int32),
            pltpu.VMEM((BE, HC), jnp.float32),
            pltpu.VMEM((128, HC), jnp.float32),
            pltpu.VMEM_SHARED((NP, HC), jnp.float32),
        ],
    )
    def k(dstx, vals, zrows, acc_out, dstv, vbuf, stage, macc):
        cid = lax.axis_index("c")
        sid = lax.axis_index("s")
        wid = _wid()
        row0 = sid * ROWS_PER_TILE
        pltpu.sync_copy(zrows, stage)
        for t in range(5):
            pltpu.sync_copy(stage, macc.at[pl.ds(row0 + t * 128, 128)])
        plsc.subcore_barrier()

        def blk(g, _):
            base = wid * epw + g * BE
            pltpu.sync_copy(dstx.at[pl.ds(base, BE)], dstv)
            pltpu.sync_copy(vals.at[pl.ds(base, BE)], vbuf)
            pltpu.sync_copy(vbuf, macc.at[dstv], add=True)
            return 0
        lax.fori_loop(0, epw // BE, blk, 0)
        plsc.subcore_barrier()

        for t in range(5):
            pltpu.sync_copy(macc.at[pl.ds(row0 + t * 128, 128)], stage)
            pltpu.sync_copy(stage,
                            acc_out.at[cid, pl.ds(row0 + t * 128, 128)])
    return k


def _pad_edges(src, dst, ea, ep):
    n = src.shape[0]
    pad = ep - n
    src = jnp.concatenate([src, jnp.full((pad,), DUMMY, jnp.int32)])
    dst = jnp.concatenate([dst, jnp.full((pad,), DUMMY, jnp.int32)])
    ea = jnp.concatenate([ea, jnp.zeros((pad,), jnp.float32)])
    return src, dst, ea


def _layer(src_x, dst_x, ea_x, epw, ep, xw, asrc16, adst16, wcv, zrows):
    alpha, tbls = _sc_pass_a(epw, ep)(src_x, dst_x, ea_x, asrc16, adst16,
                                      wcv)
    amax = _max_reduce(tbls.reshape(NWRK, 80, 1024)).reshape(MAXTBL)
    args = _sc_pass_b(epw, ep)(dst_x, alpha, amax)
    ex128 = _exp_tc(args.reshape(ep, 16))
    gath = _sc_gather(epw, ep)(src_x, xw)
    msgv = _mult_tc(gath, ex128)
    msgp = _sc_scatter_add(epw, ep)(dst_x, msgv, zrows)
    denp = _sc_scatter_add(epw, ep)(dst_x, ex128, zrows)
    return msgp, denp


def _wcv(We, ae):
    w_e = (We.reshape(H, C) * ae[0]).sum(axis=-1)
    return jnp.concatenate([w_e, jnp.zeros((8,), jnp.float32)])


def kernel(x, edge_index, edge_attr, batch, W1, as1, ad1, We1, ae1, bb1,
           W2, as2, ad2, We2, ae2, bb2, W3, as3, ad3, We3, ae3, bb3,
           fw1, fb1, fw2, fb2):
    src = edge_index[0]
    dst = edge_index[1]
    ea = edge_attr[:, 0]

    src1, dst1, ea1 = _pad_edges(src, dst, ea, EP1)
    zrows = jnp.zeros((128, HC), jnp.float32)

    eam = jnp.concatenate(
        [ea1[:, None], jnp.ones((EP1, 1), jnp.float32),
         jnp.zeros((EP1, HC - 2), jnp.float32)], axis=1)
    lsp = _sc_scatter_add(EPW1, EP1)(dst1, eam, zrows)
    st = lsp[0] + lsp[1]
    loop_attr = st[:N, 0] / jnp.maximum(st[:N, 1], 1.0)
    loop = jnp.arange(N, dtype=jnp.int32)
    src23, dst23, ea23 = _pad_edges(
        jnp.concatenate([src, loop]),
        jnp.concatenate([dst, loop]),
        jnp.concatenate([ea, loop_attr]), EP23)

    xpad = jnp.pad(x, ((0, NP - N), (0, 0)))
    xw, s16, d16 = _prep_l1(xpad, W1, _sel128(as1), _sel128(ad1))
    msgp, denp = _layer(src1, dst1, ea1, EPW1, EP1, xw, s16, d16,
                        _wcv(We1, ae1), zrows)

    xw, s16, d16 = _prep_l23(msgp, denp, bb1.reshape(1, HC), W2,
                             _sel128(as2), _sel128(ad2))
    msgp, denp = _layer(src23, dst23, ea23, EPW23, EP23, xw, s16, d16,
                        _wcv(We2, ae2), zrows)

    xw, s16, d16 = _prep_l23(msgp, denp, bb2.reshape(1, HC), W3,
                             _sel128(as3), _sel128(ad3))
    msgp, denp = _layer(src23, dst23, ea23, EPW23, EP23, xw, s16, d16,
                        _wcv(We3, ae3), zrows)

    batch2d = jnp.pad(batch, (0, NP - N),
                      constant_values=G).reshape(NP, 1)
    _, outg = _final(msgp, denp, bb3.reshape(1, HC), batch2d, fw1,
                     fb1.reshape(1, G), fw2)
    return outg[:, :1] + fb2


# speedup vs baseline: 1.3763x; 1.3763x over previous
import functools

import jax
import jax.numpy as jnp
from jax import lax
from jax.experimental import pallas as pl
from jax.experimental.pallas import tpu as pltpu
from jax.experimental.pallas import tpu_sc as plsc

N = 10000
NP = 10240
H = 8
C = 16
HC = 128
G = 64
E = 320000
NC = 2
NS = 16
NWRK = NC * NS
BE = 128
DUMMY = 10100
ROWS_PER_TILE = NP // NS
MAXTBL = NP * H

EPW1 = 10112
EP1 = EPW1 * NWRK
EPW23 = 10368
EP23 = EPW23 * NWRK

_HIGHEST = lax.Precision.HIGHEST


def _dotH(a, b, dims):
    return lax.dot_general(a, b, (dims, ((), ())),
                           precision=_HIGHEST,
                           preferred_element_type=jnp.float32)


def _prep_common(xw, sel):
    return _dotH(xw, sel, ((1,), (0,)))


def _sel128(att):
    a = att.reshape(HC)
    k = jnp.arange(HC)
    oh = (k[:, None] // C == jnp.arange(HC)[None, :]).astype(jnp.float32)
    return oh * a[:, None]


def _prep_l1(xpad, W, s_sel, d_sel):
    BM = 1280
    def body(x_ref, w_ref, ss_ref, ds_ref, xw_ref, s_ref, d_ref):
        xw = _dotH(x_ref[...], w_ref[...], ((1,), (0,)))
        xw_ref[...] = xw
        s_ref[...] = _prep_common(xw, ss_ref[...])
        d_ref[...] = _prep_common(xw, ds_ref[...])
    return pl.pallas_call(
        body,
        grid=(NP // BM,),
        in_specs=[pl.BlockSpec((BM, HC), lambda i: (i, 0)),
                  pl.BlockSpec((HC, HC), lambda i: (0, 0)),
                  pl.BlockSpec((HC, HC), lambda i: (0, 0)),
                  pl.BlockSpec((HC, HC), lambda i: (0, 0))],
        out_specs=[pl.BlockSpec((BM, HC), lambda i: (i, 0)),
                   pl.BlockSpec((BM, HC), lambda i: (i, 0)),
                   pl.BlockSpec((BM, HC), lambda i: (i, 0))],
        out_shape=[jax.ShapeDtypeStruct((NP, HC), jnp.float32),
                   jax.ShapeDtypeStruct((NP, HC), jnp.float32),
                   jax.ShapeDtypeStruct((NP, HC), jnp.float32)],
    )(xpad, W, s_sel, d_sel)


def _epilogue_h(m_ref, d_ref, b_ref, i, BM):
    msum = m_ref[0] + m_ref[1]
    dfull = d_ref[0] + d_ref[1]
    hval = jax.nn.relu(msum / (dfull + 1e-16) + b_ref[...])
    row = jax.lax.broadcasted_iota(jnp.int32, (BM, HC), 0) + i * BM
    return jnp.where(row < N, hval, 0.0)


def _prep_l23(msgp, denp, bias2d, W, s_sel, d_sel):
    BM = 1280
    def body(m_ref, d_ref, b_ref, w_ref, ss_ref, ds_ref,
             xw_ref, s_ref, d16_ref):
        i = pl.program_id(0)
        hval = _epilogue_h(m_ref, d_ref, b_ref, i, BM)
        xw = _dotH(hval, w_ref[...], ((1,), (0,)))
        xw_ref[...] = xw
        s_ref[...] = _prep_common(xw, ss_ref[...])
        d16_ref[...] = _prep_common(xw, ds_ref[...])
    return pl.pallas_call(
        body,
        grid=(NP // BM,),
        in_specs=[pl.BlockSpec((2, BM, HC), lambda i: (0, i, 0)),
                  pl.BlockSpec((2, BM, HC), lambda i: (0, i, 0)),
                  pl.BlockSpec((1, HC), lambda i: (0, 0)),
                  pl.BlockSpec((HC, HC), lambda i: (0, 0)),
                  pl.BlockSpec((HC, HC), lambda i: (0, 0)),
                  pl.BlockSpec((HC, HC), lambda i: (0, 0))],
        out_specs=[pl.BlockSpec((BM, HC), lambda i: (i, 0)),
                   pl.BlockSpec((BM, HC), lambda i: (i, 0)),
                   pl.BlockSpec((BM, HC), lambda i: (i, 0))],
        out_shape=[jax.ShapeDtypeStruct((NP, HC), jnp.float32),
                   jax.ShapeDtypeStruct((NP, HC), jnp.float32),
                   jax.ShapeDtypeStruct((NP, HC), jnp.float32)],
    )(msgp, denp, bias2d, W, s_sel, d_sel)


def _max_reduce(tables3d):
    def body(t_ref, o_ref):
        m = jnp.max(t_ref[...], axis=0)
        o_ref[...] = jnp.where(m < -1e29, 0.0, m)
    BR = 8
    return pl.pallas_call(
        body,
        grid=(80 // BR,),
        in_specs=[pl.BlockSpec((NWRK, BR, 1024), lambda i: (0, i, 0))],
        out_specs=pl.BlockSpec((BR, 1024), lambda i: (i, 0)),
        out_shape=jax.ShapeDtypeStruct((80, 1024), jnp.float32),
    )(tables3d)


def _exp_tc(args16):
    EPL = args16.shape[0]
    BM = 4096
    k = jnp.arange(HC)

    def body(a_ref, e_ref, o_ref):
        ex = jnp.exp(a_ref[...])
        o_ref[...] = _dotH(ex, e_ref[...], ((1,), (0,)))

    expand = (jnp.arange(16)[:, None] == (k[None, :] // C)).astype(
        jnp.float32)
    return pl.pallas_call(
        body,
        grid=(EPL // BM,),
        in_specs=[pl.BlockSpec((BM, 16), lambda i: (i, 0)),
                  pl.BlockSpec((16, HC), lambda i: (0, 0))],
        out_specs=pl.BlockSpec((BM, HC), lambda i: (i, 0)),
        out_shape=jax.ShapeDtypeStruct((EPL, HC), jnp.float32),
    )(args16, expand)


def _mult_tc(a, b):
    EPL = a.shape[0]
    BM = 4096
    def body(a_ref, b_ref, o_ref):
        o_ref[...] = a_ref[...] * b_ref[...]
    return pl.pallas_call(
        body,
        grid=(EPL // BM,),
        in_specs=[pl.BlockSpec((BM, HC), lambda i: (i, 0)),
                  pl.BlockSpec((BM, HC), lambda i: (i, 0))],
        out_specs=pl.BlockSpec((BM, HC), lambda i: (i, 0)),
        out_shape=jax.ShapeDtypeStruct((EPL, HC), jnp.float32),
    )(a, b)


def _final(msgp, denp, bias2d, batch2d, fw1, fb1r, fw2):
    BM = 1280
    def body(m_ref, d_ref, b_ref, bt_ref, f1_ref, f1b_ref, f2_ref,
             gs_ref, o_ref):
        i = pl.program_id(0)
        hval = _epilogue_h(m_ref, d_ref, b_ref, i, BM)
        onehot = (bt_ref[...] == jnp.arange(G)[None, :]).astype(jnp.float32)
        part = _dotH(onehot, hval, ((0,), (0,)))
        cnt = _dotH(onehot, jnp.ones((BM, HC), jnp.float32), ((0,), (0,)))

        @pl.when(i == 0)
        def _():
            gs_ref[...] = jnp.zeros((2, G, HC), jnp.float32)

        gs_ref[0, :, :] = gs_ref[0, :, :] + part
        gs_ref[1, :, :] = gs_ref[1, :, :] + cnt

        @pl.when(i == NP // BM - 1)
        def _():
            ge = gs_ref[0, :, :] / jnp.maximum(gs_ref[1, :, :], 1.0)
            hid = jax.nn.relu(_dotH(ge, f1_ref[...], ((1,), (1,)))
                              + f1b_ref[...])
            out = _dotH(hid, f2_ref[...], ((1,), (1,)))
            o_ref[...] = jnp.broadcast_to(out, (G, HC))
    return pl.pallas_call(
        body,
        grid=(NP // BM,),
        in_specs=[pl.BlockSpec((2, BM, HC), lambda i: (0, i, 0)),
                  pl.BlockSpec((2, BM, HC), lambda i: (0, i, 0)),
                  pl.BlockSpec((1, HC), lambda i: (0, 0)),
                  pl.BlockSpec((BM, 1), lambda i: (i, 0)),
                  pl.BlockSpec((G, HC), lambda i: (0, 0)),
                  pl.BlockSpec((1, G), lambda i: (0, 0)),
                  pl.BlockSpec((1, G), lambda i: (0, 0))],
        out_specs=[pl.BlockSpec((2, G, HC), lambda i: (0, 0, 0)),
                   pl.BlockSpec((G, HC), lambda i: (0, 0))],
        out_shape=[jax.ShapeDtypeStruct((2, G, HC), jnp.float32),
                   jax.ShapeDtypeStruct((G, HC), jnp.float32)],
    )(msgp, denp, bias2d, batch2d, fw1, fb1r, fw2)


_MESH = plsc.VectorSubcoreMesh(core_axis_name="c", subcore_axis_name="s")


def _wid():
    return lax.axis_index("s") * NC + lax.axis_index("c")


def _zero_zbuf(z_ref):
    zv = jnp.zeros((16,), jnp.float32)
    def zrow(r, _):
        for j in range(8):
            z_ref[r, pl.ds(16 * j, 16)] = zv
        return 0
    lax.fori_loop(0, 128, zrow, 0)


def _sc_pass_a(epw, ep):
    @functools.partial(
        pl.kernel, mesh=_MESH,
        compiler_params=pltpu.CompilerParams(needs_layout_passes=False),
        out_type=[jax.ShapeDtypeStruct((ep * 16,), jnp.float32),
                  jax.ShapeDtypeStruct((NWRK, MAXTBL), jnp.float32)],
        scratch_types=[
            pltpu.VMEM((BE,), jnp.int32),
            pltpu.VMEM((BE + 16,), jnp.int32),
            pltpu.VMEM((BE + 16,), jnp.float32),
            pltpu.VMEM((BE, HC), jnp.float32),
            pltpu.VMEM((BE, HC), jnp.float32),
            pltpu.VMEM((BE * 16,), jnp.float32),
            pltpu.VMEM((16,), jnp.float32),
            pltpu.VMEM((MAXTBL + 16,), jnp.float32),
            pltpu.SemaphoreType.DMA,
            pltpu.SemaphoreType.DMA,
        ],
    )
    def k(srcx, dstx, eax, asrc_t, adst_t, wcv, alpha_out, tbl_out,
          srcv, dstv, eav, asg, adg, alb, wcvv, mtbl, sem1, sem2):
        wid = _wid()
        pltpu.sync_copy(wcv, wcvv)
        wv = wcvv[...]
        cv = jnp.where(lax.iota(jnp.int32, 16) < 8, 0.0, -1e30)

        neg = jnp.full((16,), -1e30, jnp.float32)
        def init(r, _):
            mtbl[pl.ds(r * 16, 16)] = neg
            return 0
        lax.fori_loop(0, (MAXTBL + 16) // 16, init, 0)

        lane = lax.iota(jnp.int32, 16)
        spill = MAXTBL + lane

        def blk(g, _):
            base = wid * epw + g * BE
            pltpu.sync_copy(srcx.at[pl.ds(base, BE)], srcv)
            pltpu.sync_copy(dstx.at[pl.ds(base, BE)],
                            dstv.at[pl.ds(0, BE)])
            pltpu.sync_copy(eax.at[pl.ds(base, BE)], eav.at[pl.ds(0, BE)])
            cp1 = pltpu.async_copy(asrc_t.at[srcv], asg, sem1)
            cp2 = pltpu.async_copy(adst_t.at[dstv.at[pl.ds(0, BE)]], adg,
                                   sem2)
            cp1.wait()
            cp2.wait()

            def edge(b, _):
                ea = eav[pl.ds(b, 16)][0]
                a = ((asg[b, pl.ds(0, 16)] + adg[b, pl.ds(0, 16)])
                     + (ea * wv + cv))
                a = jnp.maximum(a, 0.0) + 0.2 * jnp.minimum(a, 0.0)
                alb[pl.ds(b * 16, 16)] = a
                d0 = dstv[pl.ds(b, 16)][0]
                idx = jnp.where(lane < 8, d0 * H + lane, spill)
                cur = plsc.load_gather(mtbl, [idx])
                plsc.store_scatter(mtbl, [idx], jnp.maximum(cur, a))
                return 0
            lax.fori_loop(0, BE, edge, 0)
            pltpu.sync_copy(alb, alpha_out.at[pl.ds(base * 16, BE * 16)])
            return 0
        lax.fori_loop(0, epw // BE, blk, 0)
        pltpu.sync_copy(mtbl.at[pl.ds(0, MAXTBL)], tbl_out.at[wid])
    return k


def _sc_pass_b(epw, ep):
    @functools.partial(
        pl.kernel, mesh=_MESH,
        compiler_params=pltpu.CompilerParams(needs_layout_passes=False),
        out_type=jax.ShapeDtypeStruct((ep * 16,), jnp.float32),
        scratch_types=[
            pltpu.VMEM((BE + 16,), jnp.int32),
            pltpu.VMEM((BE * 16,), jnp.float32),
            pltpu.VMEM((MAXTBL,), jnp.float32),
        ],
    )
    def k(dstx, alpha_in, amax_f, args_out, dstv, alb, amv):
        wid = _wid()
        pltpu.sync_copy(amax_f, amv)
        lane7 = jnp.bitwise_and(lax.iota(jnp.int32, 16), 7)

        def blk(g, _):
            base = wid * epw + g * BE
            pltpu.sync_copy(dstx.at[pl.ds(base, BE)],
                            dstv.at[pl.ds(0, BE)])
            pltpu.sync_copy(alpha_in.at[pl.ds(base * 16, BE * 16)], alb)

            def edge(b, _):
                d0 = dstv[pl.ds(b, 16)][0]
                idx = d0 * H + lane7
                am = plsc.load_gather(amv, [idx])
                alb[pl.ds(b * 16, 16)] = alb[pl.ds(b * 16, 16)] - am
                return 0
            lax.fori_loop(0, BE, edge, 0)
            pltpu.sync_copy(alb, args_out.at[pl.ds(base * 16, BE * 16)])
            return 0
        lax.fori_loop(0, epw // BE, blk, 0)
    return k


def _sc_gather(epw, ep):
    @functools.partial(
        pl.kernel, mesh=_MESH,
        out_type=jax.ShapeDtypeStruct((ep, HC), jnp.float32),
        scratch_types=[
            pltpu.VMEM((BE,), jnp.int32),
            pltpu.VMEM((BE, HC), jnp.float32),
            pltpu.SemaphoreType.DMA,
        ],
    )
    def k(srcx, xw_t, out, srcv, xwg, sem):
        wid = _wid()

        def blk(g, _):
            base = wid * epw + g * BE
            pltpu.sync_copy(srcx.at[pl.ds(base, BE)], srcv)
            pltpu.async_copy(xw_t.at[srcv], xwg, sem).wait()
            pltpu.sync_copy(xwg, out.at[pl.ds(base, BE)])
            return 0
        lax.fori_loop(0, epw // BE, blk, 0)
    return k


def _sc_scatter_add(epw, ep):
    @functools.partial(
        pl.kernel, mesh=_MESH,
        out_type=jax.ShapeDtypeStruct((NC, NP, HC), jnp.float32),
        scratch_types=[
            pltpu.VMEM((BE,), jnp.int32),
            pltpu.VMEM((BE, HC), jnp.float32),
            pltpu.VMEM((128, HC), jnp.float32),
            pltpu.VMEM_SHARED((NP, HC), jnp.float32),
        ],
    )
    def k(dstx, vals, zrows, acc_out, dstv, vbuf, stage, macc):
        cid = lax.axis_index("c")
        sid = lax.axis_index("s")
        wid = _wid()
        row0 = sid * ROWS_PER_TILE
        pltpu.sync_copy(zrows, stage)
        for t in range(5):
            pltpu.sync_copy(stage, macc.at[pl.ds(row0 + t * 128, 128)])
        plsc.subcore_barrier()

        def blk(g, _):
            base = wid * epw + g * BE
            pltpu.sync_copy(dstx.at[pl.ds(base, BE)], dstv)
            pltpu.sync_copy(vals.at[pl.ds(base, BE)], vbuf)
            pltpu.sync_copy(vbuf, macc.at[dstv], add=True)
            return 0
        lax.fori_loop(0, epw // BE, blk, 0)
        plsc.subcore_barrier()

        for t in range(5):
            pltpu.sync_copy(macc.at[pl.ds(row0 + t * 128, 128)], stage)
            pltpu.sync_copy(stage,
                            acc_out.at[cid, pl.ds(row0 + t * 128, 128)])
    return k


def _pad_edges(src, dst, ea, ep):
    n = src.shape[0]
    pad = ep - n
    src = jnp.concatenate([src, jnp.full((pad,), DUMMY, jnp.int32)])
    dst = jnp.concatenate([dst, jnp.full((pad,), DUMMY, jnp.int32)])
    ea = jnp.concatenate([ea, jnp.zeros((pad,), jnp.float32)])
    return src, dst, ea


def _layer(src_x, dst_x, ea_x, epw, ep, xw, asrc16, adst16, wcv, zrows):
    alpha, tbls = _sc_pass_a(epw, ep)(src_x, dst_x, ea_x, asrc16, adst16,
                                      wcv)
    amax = _max_reduce(tbls.reshape(NWRK, 80, 1024)).reshape(MAXTBL)
    args = _sc_pass_b(epw, ep)(dst_x, alpha, amax)
    ex128 = _exp_tc(args.reshape(ep, 16))
    gath = _sc_gather(epw, ep)(src_x, xw)
    msgv = _mult_tc(gath, ex128)
    msgp = _sc_scatter_add(epw, ep)(dst_x, msgv, zrows)
    denp = _sc_scatter_add(epw, ep)(dst_x, ex128, zrows)
    return msgp, denp


def _wcv(We, ae):
    w_e = (We.reshape(H, C) * ae[0]).sum(axis=-1)
    return jnp.concatenate([w_e, jnp.zeros((8,), jnp.float32)])


def kernel(x, edge_index, edge_attr, batch, W1, as1, ad1, We1, ae1, bb1,
           W2, as2, ad2, We2, ae2, bb2, W3, as3, ad3, We3, ae3, bb3,
           fw1, fb1, fw2, fb2):
    src = edge_index[0]
    dst = edge_index[1]
    ea = edge_attr[:, 0]

    src1, dst1, ea1 = _pad_edges(src, dst, ea, EP1)
    zrows = jnp.zeros((128, HC), jnp.float32)

    eam = jnp.concatenate(
        [ea1[:, None], jnp.ones((EP1, 1), jnp.float32),
         jnp.zeros((EP1, HC - 2), jnp.float32)], axis=1)
    lsp = _sc_scatter_add(EPW1, EP1)(dst1, eam, zrows)
    st = lsp[0] + lsp[1]
    loop_attr = st[:N, 0] / jnp.maximum(st[:N, 1], 1.0)
    loop = jnp.arange(N, dtype=jnp.int32)
    src23, dst23, ea23 = _pad_edges(
        jnp.concatenate([src, loop]),
        jnp.concatenate([dst, loop]),
        jnp.concatenate([ea, loop_attr]), EP23)

    xpad = jnp.pad(x, ((0, NP - N), (0, 0)))
    xw, s16, d16 = _prep_l1(xpad, W1, _sel128(as1), _sel128(ad1))
    msgp, denp = _layer(src1, dst1, ea1, EPW1, EP1, xw, s16, d16,
                        _wcv(We1, ae1), zrows)

    xw, s16, d16 = _prep_l23(msgp, denp, bb1.reshape(1, HC), W2,
                             _sel128(as2), _sel128(ad2))
    msgp, denp = _layer(src23, dst23, ea23, EPW23, EP23, xw, s16, d16,
                        _wcv(We2, ae2), zrows)

    xw, s16, d16 = _prep_l23(msgp, denp, bb2.reshape(1, HC), W3,
                             _sel128(as3), _sel128(ad3))
    msgp, denp = _layer(src23, dst23, ea23, EPW23, EP23, xw, s16, d16,
                        _wcv(We3, ae3), zrows)

    batch2d = jnp.pad(batch, (0, NP - N),
                      constant_values=G).reshape(NP, 1)
    _, outg = _final(msgp, denp, bb3.reshape(1, HC), batch2d, fw1,
                     fb1.reshape(1, G), fw2)
    return outg[:, :1] + fb2
